# baseline (speedup 1.0000x reference)
import functools

import jax
import jax.numpy as jnp
from jax import lax
from jax.experimental import pallas as pl
from jax.experimental.pallas import tpu as pltpu
from jax.experimental.pallas import tpu_sc as plsc

_N_CAT = 26
_VOCAB = 100000
_DIM = 16
_N_REG = 4
_ENC = 64
_B = 1024
_L = 200
_NPLANE = _N_CAT * _DIM
_OUT_D = _ENC + _NPLANE + _N_REG
_NW = 32
_PPW = _NPLANE // _NW
_NLT = _L // 8


_FPC = _N_CAT // 2


def _sc_gather(tabT, idxT):
    mesh = plsc.VectorSubcoreMesh(core_axis_name="c", subcore_axis_name="s")

    @functools.partial(
        pl.kernel,
        out_type=jax.ShapeDtypeStruct((_OUT_D, _L, 1024), jnp.float32),
        mesh=mesh,
        scratch_types=[
            pltpu.VMEM((_VOCAB,), jnp.float32),
            pltpu.VMEM((8, 512), jnp.int32),
            pltpu.VMEM((8, 512), jnp.int32),
            pltpu.VMEM((8, 512), jnp.float32),
            pltpu.VMEM((8, 512), jnp.float32),
            pltpu.VMEM_SHARED((_L, 1024), jnp.int32),
            pltpu.SemaphoreType.DMA,
            pltpu.SemaphoreType.DMA,
            pltpu.SemaphoreType.DMA,
            pltpu.SemaphoreType.DMA,
            pltpu.SemaphoreType.DMA,
            pltpu.SemaphoreType.DMA,
        ],
        compiler_params=pltpu.CompilerParams(
            use_tc_tiling_on_sc=True, needs_layout_passes=False
        ),
    )
    def k(tab_hbm, idx_hbm, out_hbm, plane_v, ib0, ib1, vb0, vb1, stage_v,
          sp, si0, si1, so0, so1, ss):
        core = lax.axis_index("c")
        s = lax.axis_index("s")
        d = s

        def stage_start(fi):
            f = core * _FPC + fi
            pltpu.async_copy(
                idx_hbm.at[f, pl.ds(s * 8, 8), :],
                stage_v.at[pl.ds(s * 8, 8), :],
                ss,
            )

            @pl.when(s < _NLT - 16)
            def _():
                pltpu.async_copy(
                    idx_hbm.at[f, pl.ds((s + 16) * 8, 8), :],
                    stage_v.at[pl.ds((s + 16) * 8, 8), :],
                    ss,
                )

        def stage_wait():
            pltpu.make_async_copy(
                idx_hbm.at[0, pl.ds(0, 8), :],
                stage_v.at[pl.ds(0, 8), :],
                ss,
            ).wait()

            @pl.when(s < _NLT - 16)
            def _():
                pltpu.make_async_copy(
                    idx_hbm.at[0, pl.ds(0, 8), :],
                    stage_v.at[pl.ds(0, 8), :],
                    ss,
                ).wait()

        def gather_chunk(ib, vb):
            lag = 3

            @pl.loop(0, 8)
            def _(r):
                g = []
                for j in range(512 // 16):
                    iv = ib[r, pl.ds(j * 16, 16)]
                    g.append(plsc.load_gather(plane_v, [iv]))
                    if j >= lag:
                        vb[r, pl.ds((j - lag) * 16, 16)] = g[j - lag]
                for j in range(512 // 16 - lag, 512 // 16):
                    vb[r, pl.ds(j * 16, 16)] = g[j]

        def idx_src(lt, h):
            return stage_v.at[pl.ds(lt * 8, 8), pl.ds(h * 512, 512)]

        def wait_idx(ib, si):
            pltpu.make_async_copy(idx_src(0, 0), ib, si).wait()

        def wait_out(vb, so):
            pltpu.make_async_copy(
                out_hbm.at[0, pl.ds(0, 8), pl.ds(0, 512)], vb, so
            ).wait()

        @pl.loop(0, _FPC)
        def _(fi):
            f = core * _FPC + fi
            c = _ENC + f * _DIM + d
            stage_start(fi)
            pltpu.async_copy(tab_hbm.at[f, d, :], plane_v, sp)
            stage_wait()
            plsc.subcore_barrier()
            pltpu.make_async_copy(tab_hbm.at[0, 0, :], plane_v, sp).wait()
            pltpu.async_copy(idx_src(0, 0), ib0, si0)

            @pl.loop(0, _NLT)
            def _(lt):
                wait_idx(ib0, si0)
                pltpu.async_copy(idx_src(lt, 1), ib1, si1)

                @pl.when(lt > 0)
                def _():
                    wait_out(vb0, so0)

                gather_chunk(ib0, vb0)
                pltpu.async_copy(
                    vb0,
                    out_hbm.at[c, pl.ds(lt * 8, 8), pl.ds(0, 512)],
                    so0,
                )
                wait_idx(ib1, si1)
                nlt = jnp.minimum(lt + 1, _NLT - 1)
                pltpu.async_copy(idx_src(nlt, 0), ib0, si0)

                @pl.when(lt > 0)
                def _():
                    wait_out(vb1, so1)

                gather_chunk(ib1, vb1)
                pltpu.async_copy(
                    vb1,
                    out_hbm.at[c, pl.ds(lt * 8, 8), pl.ds(512, 512)],
                    so1,
                )

            wait_idx(ib0, si0)
            wait_out(vb0, so0)
            wait_out(vb1, so1)
            plsc.subcore_barrier()

    return k(tabT, idxT)


def _tc_fill_base(outP, baseT):
    def body(b_ref, o_ref, out_ref):
        del o_ref
        out_ref[...] = jnp.transpose(b_ref[...], (1, 0, 2))

    return pl.pallas_call(
        body,
        grid=(_NLT,),
        in_specs=[
            pl.BlockSpec((8, _ENC, 1024), lambda i: (i, 0, 0)),
            pl.BlockSpec((_ENC, 8, 1024), lambda i: (0, i, 0)),
        ],
        out_specs=pl.BlockSpec((_ENC, 8, 1024), lambda i: (0, i, 0)),
        out_shape=jax.ShapeDtypeStruct((_OUT_D, _L, 1024), jnp.float32),
        input_output_aliases={1: 0},
    )(baseT, outP)


def _tc_fill_reg(outP, regT):
    def body(r_ref, o_ref, out_ref):
        del o_ref
        out_ref[...] = jnp.transpose(r_ref[...], (1, 0, 2))

    c0 = (_ENC + _NPLANE) // 4

    return pl.pallas_call(
        body,
        grid=(_NLT,),
        in_specs=[
            pl.BlockSpec((8, _N_REG, 1024), lambda i: (i, 0, 0)),
            pl.BlockSpec((_N_REG, 8, 1024), lambda i: (c0, i, 0)),
        ],
        out_specs=pl.BlockSpec((_N_REG, 8, 1024), lambda i: (c0, i, 0)),
        out_shape=jax.ShapeDtypeStruct((_OUT_D, _L, 1024), jnp.float32),
        input_output_aliases={1: 0},
    )(regT, outP)


def kernel(cat_indices, reg_feats, base_out, tables):
    tabT = jnp.transpose(tables, (0, 2, 1))
    idxT = jnp.transpose(cat_indices, (2, 1, 0))
    baseT = jnp.transpose(base_out, (1, 2, 0))
    regT = jnp.transpose(reg_feats, (1, 2, 0))
    outP = _sc_gather(tabT, idxT)
    outP = _tc_fill_base(outP, baseT)
    outP = _tc_fill_reg(outP, regT)
    return jnp.transpose(outP, (2, 1, 0))
